# baseline (speedup 1.0000x reference)
import functools

import jax
import jax.numpy as jnp
from jax import lax
from jax.experimental import pallas as pl
from jax.experimental.pallas import tpu as pltpu
from jax.experimental.pallas import tpu_sc as plsc

S = 8000
LG = 2000
F = 64
NC, NS = 2, 16
NW = NC * NS
CPT = 5120
CAP = NW * CPT
SPAD = 8192
LPAD = 2048
EBLK = 2048
STS = 400
GCH = 128

def _mesh():
    return plsc.VectorSubcoreMesh(core_axis_name="c", subcore_axis_name="s")


def _sc_params():
    return pltpu.CompilerParams(use_tc_tiling_on_sc=False)


def _lig_prologue_body(lp_ref, t_ref, tw1, tb1, tw2, tb2, wl, bl, wg, bg, wb,
                       out_ref):
    t = t_ref[...]
    half = 32
    k = lax.broadcasted_iota(jnp.int32, (1, half), 1).astype(jnp.float32)
    freqs = jnp.exp(-jnp.log(10000.0) / (half - 1) * k)
    a = t * freqs
    ht = jnp.concatenate([jnp.sin(a), jnp.cos(a)], axis=1)
    x = jnp.dot(ht, tw1[...], preferred_element_type=jnp.float32) + tb1[...]
    c = 0.7978845608028654
    g = 0.5 * x * (1.0 + jnp.tanh(c * (x + 0.044715 * x * x * x)))
    ht = jnp.dot(g, tw2[...], preferred_element_type=jnp.float32) + tb2[...]
    lp = lp_ref[...]
    base = jnp.dot(lp, wl[...], preferred_element_type=jnp.float32) + bl[...]
    gate = jax.nn.sigmoid(
        jnp.dot(ht, wg[...], preferred_element_type=jnp.float32) + bg[...])
    out_ref[...] = base * gate + jnp.dot(ht, wb[...],
                                         preferred_element_type=jnp.float32)


def _lig_prologue(lp, t, p):
    tm, c = p['time_mlp'], p['csl']
    full = lambda s: pl.BlockSpec(s, lambda: (0,) * len(s))
    args = (lp, t,
            tm['w1'], tm['b1'].reshape(1, -1), tm['w2'], tm['b2'].reshape(1, -1),
            c['wl'], c['bl'].reshape(1, -1), c['wg'], c['bg'].reshape(1, -1),
            c['wb'])
    return pl.pallas_call(
        _lig_prologue_body,
        out_shape=jax.ShapeDtypeStruct((LG, F), jnp.float32),
        in_specs=[full(a.shape) for a in args],
        out_specs=full((LG, F)),
    )(*args)


def _ln(x, g, b):
    m = jnp.mean(x, axis=-1, keepdims=True)
    v = jnp.mean((x - m) ** 2, axis=-1, keepdims=True)
    return (x - m) * jax.lax.rsqrt(v + 1e-5) * g + b


def _surf_mask_body(sp_ref, bs_ref, lpt_ref, bl_ref, sw, sb,
                    nw1, nb1, nw2, nb2, ng, nbl,
                    mask_ref, hs0_ref, hs1_ref, hs2_ref):
    sp = sp_ref[...]
    d2 = jnp.zeros((STS, LG), jnp.float32)
    for ci in range(3):
        diff = sp[:, ci:ci + 1] - lpt_ref[ci:ci + 1, :]
        d2 = d2 + diff * diff
    same = bs_ref[...] == bl_ref[...]
    mask_ref[...] = jnp.where(same & (d2 < 9.0), jnp.int32(1), jnp.int32(0))

    hs = jnp.dot(sp, sw[...], preferred_element_type=jnp.float32) + sb[...]
    hs0_ref[...] = hs
    outs = (hs1_ref, hs2_ref)
    for li in range(2):
        up = jnp.maximum(
            jnp.dot(hs, nw1[li], preferred_element_type=jnp.float32) + nb1[li],
            0.0)
        up = jnp.dot(up, nw2[li], preferred_element_type=jnp.float32) + nb2[li]
        hs = hs + _ln(up, ng[li], nbl[li])
        outs[li][...] = hs


def _surf_mask(sp, bs, lp, bl, p):
    nw1 = jnp.stack([cv['node']['w1'][:F] for cv in p['convs'][:2]])
    nb1 = jnp.stack([cv['node']['b1'].reshape(1, -1) for cv in p['convs'][:2]])
    nw2 = jnp.stack([cv['node']['w2'] for cv in p['convs'][:2]])
    nb2 = jnp.stack([cv['node']['b2'].reshape(1, -1) for cv in p['convs'][:2]])
    ng = jnp.stack([cv['node']['ln_g'].reshape(1, -1) for cv in p['convs'][:2]])
    nbl = jnp.stack([cv['node']['ln_b'].reshape(1, -1) for cv in p['convs'][:2]])
    grid = S // STS
    tile = lambda s: pl.BlockSpec(s, lambda i: (i,) + (0,) * (len(s) - 1))
    full = lambda s: pl.BlockSpec(s, lambda i: (0,) * len(s))
    args = (sp, bs.reshape(S, 1), lp.T, bl.reshape(1, LG),
            p['surf_enc']['w'], p['surf_enc']['b'].reshape(1, -1),
            nw1, nb1, nw2, nb2, ng, nbl)
    in_specs = [tile((STS, 3)), tile((STS, 1)), full((3, LG)), full((1, LG))]
    in_specs += [full(a.shape) for a in args[4:]]
    return pl.pallas_call(
        _surf_mask_body,
        grid=(grid,),
        out_shape=[jax.ShapeDtypeStruct((S, LG), jnp.int32)] +
                  [jax.ShapeDtypeStruct((S, F), jnp.float32)] * 3,
        in_specs=in_specs,
        out_specs=[tile((STS, LG))] + [tile((STS, F))] * 3,
    )(*args)


def _sc_gather(table, idx3):
    samp = table.shape[1:]
    dt = table.dtype
    rb = dt.itemsize
    for d in samp:
        rb *= d
    iters = CPT // GCH
    NB = max(2, min(8, 400 * 1024 // (GCH * rb)))

    @functools.partial(
        pl.kernel, mesh=_mesh(),
        out_type=jax.ShapeDtypeStruct((CAP,) + samp, dt),
        scratch_types=[pltpu.VMEM((CPT // GCH, GCH), jnp.int32),
                       pltpu.VMEM((NB, GCH) + samp, dt)] +
                      [pltpu.SemaphoreType.DMA] * (2 * NB))
    def k(tab_hbm, idx_hbm, out_hbm, idx_v, rows_v, *sems):
        gsem, wsem = sems[:NB], sems[NB:]
        wid = lax.axis_index("s") * NC + lax.axis_index("c")
        base = wid * CPT
        pltpu.sync_copy(idx_hbm.at[wid], idx_v)

        gh = [None] * NB
        wh = [None] * NB
        for i in range(iters + NB - 1):
            if i < iters:
                b = i % NB
                if wh[b] is not None:
                    wh[b].wait()
                gh[b] = pltpu.async_copy(
                    tab_hbm.at[idx_v.at[i]], rows_v.at[b], gsem[b])
            j = i - (NB - 1)
            if j >= 0:
                bj = j % NB
                gh[bj].wait()
                wh[bj] = pltpu.async_copy(
                    rows_v.at[bj], out_hbm.at[pl.ds(base + j * GCH, GCH)],
                    wsem[bj])
        for h in wh:
            if h is not None:
                h.wait()

    return k(table, idx3)


def _sc_scatter_add(vals, dst3, zeros):
    iters = CPT // GCH
    NB = 8
    dt = jnp.bfloat16

    @functools.partial(
        pl.kernel, mesh=_mesh(), compiler_params=_sc_params(),
        out_type=jax.ShapeDtypeStruct((NC, LPAD, F), dt),
        scratch_types=[pltpu.VMEM((CPT // GCH, GCH), jnp.int32),
                       pltpu.VMEM((NB, GCH, F), dt),
                       pltpu.VMEM_SHARED((LPAD, F), dt)] +
                      [pltpu.SemaphoreType.DMA] * (2 * NB))
    def k(v_hbm, d_hbm, z_hbm, out_hbm, idx_v, rows_v, acc_sh, *sems):
        lsem, asem = sems[:NB], sems[NB:]
        cid = lax.axis_index("c")
        sid = lax.axis_index("s")
        wid = sid * NC + cid
        base = wid * CPT
        stripe = LPAD // NS
        pltpu.sync_copy(z_hbm.at[pl.ds(sid * stripe, stripe)],
                        acc_sh.at[pl.ds(sid * stripe, stripe)])
        pltpu.sync_copy(d_hbm.at[wid], idx_v)
        plsc.subcore_barrier()

        lh = [None] * NB
        ah = [None] * NB
        for i in range(iters + NB - 1):
            if i < iters:
                b = i % NB
                if ah[b] is not None:
                    ah[b].wait()
                lh[b] = pltpu.async_copy(
                    v_hbm.at[pl.ds(base + i * GCH, GCH)], rows_v.at[b],
                    lsem[b])
            j = i - (NB - 1)
            if j >= 0:
                bj = j % NB
                lh[bj].wait()
                ah[bj] = pltpu.async_copy(
                    rows_v.at[bj], acc_sh.at[idx_v.at[j]], asem[bj],
                    add=True)
        for h in ah:
            if h is not None:
                h.wait()

        plsc.subcore_barrier()
        pltpu.sync_copy(acc_sh.at[pl.ds(sid * stripe, stripe)],
                        out_hbm.at[cid].at[pl.ds(sid * stripe, stripe)])

    return k(vals, dst3, zeros)


def _edge_mlp_common(gs, gl, he, w1, b1, w2, b2, g, b):
    bf = jnp.bfloat16
    w = w1[...]
    x = (jnp.dot(gs, w[0:F].astype(bf), preferred_element_type=jnp.float32) +
         jnp.dot(gl, w[F:2 * F].astype(bf),
                 preferred_element_type=jnp.float32) +
         jnp.dot(he.astype(bf), w[2 * F:3 * F].astype(bf),
                 preferred_element_type=jnp.float32)
         + b1[...])
    x = jnp.maximum(x, 0.0)
    x = jnp.dot(x.astype(bf), w2[...].astype(bf),
                preferred_element_type=jnp.float32) + b2[...]
    return he + _ln(x, g[...], b[...])


def _edge_mlp0_body(gsurf_ref, glig_ref, w1, b1, w2, b2, g, b, emw, off,
                    out_ref, obf_ref):
    f32 = jnp.float32
    sp = gsurf_ref[:, 0:3].astype(f32) + gsurf_ref[:, 3:6].astype(f32)
    lp = glig_ref[:, 0:3]
    ev = sp - lp
    nrm = jnp.sqrt(jnp.sum(ev * ev, axis=1, keepdims=True))
    v = ev / (nrm + 1e-7)
    coeff = -0.5 / ((10.0 / 18.0) ** 2)
    sca = jnp.exp(coeff * (nrm - off[...]) ** 2)
    em = emw[...]
    he0 = jnp.concatenate([sca] + [v[:, ci:ci + 1] * em for ci in range(3)],
                          axis=1)
    gs = gsurf_ref[:, 16:16 + F]
    gl = glig_ref[:, 16:16 + F].astype(jnp.bfloat16)
    he_new = _edge_mlp_common(gs, gl, he0, w1, b1, w2, b2, g, b)
    out_ref[...] = he_new
    obf_ref[...] = he_new.astype(jnp.bfloat16)


def _edge_mlp0(gsurf, glig, cv, p):
    full = lambda s: pl.BlockSpec(s, lambda i: (0,) * len(s))
    tile = lambda s: pl.BlockSpec(s, lambda i: (i,) + (0,) * (len(s) - 1))
    e = cv['edge']
    return pl.pallas_call(
        _edge_mlp0_body,
        grid=(CAP // EBLK,),
        out_shape=[jax.ShapeDtypeStruct((CAP, F), jnp.float32),
                   jax.ShapeDtypeStruct((CAP, F), jnp.bfloat16)],
        in_specs=[tile((EBLK, 256)), tile((EBLK, 128)),
                  full((192, 128)), full((1, 128)), full((128, F)),
                  full((1, F)), full((1, F)), full((1, F)),
                  full((1, 15)), full((1, 19))],
        out_specs=[tile((EBLK, F))] * 2,
    )(gsurf, glig, e['w1'], e['b1'].reshape(1, -1), e['w2'],
      e['b2'].reshape(1, -1), e['ln_g'].reshape(1, -1),
      e['ln_b'].reshape(1, -1), p['edge_map_w'],
      p['gs_offset'].reshape(1, 19))


def _edge_mlpN_body(col, gsurf_ref, ghl_ref, he_ref, w1, b1, w2, b2, g, b,
                    out_ref, obf_ref):
    gs = gsurf_ref[:, col:col + F]
    he_new = _edge_mlp_common(gs, ghl_ref[:, 0:F].astype(jnp.bfloat16),
                              he_ref[...],
                              w1, b1, w2, b2, g, b)
    out_ref[...] = he_new
    obf_ref[...] = he_new.astype(jnp.bfloat16)


def _edge_mlpN(gsurf, ghl, he, cv, col):
    full = lambda s: pl.BlockSpec(s, lambda i: (0,) * len(s))
    tile = lambda s: pl.BlockSpec(s, lambda i: (i,) + (0,) * (len(s) - 1))
    e = cv['edge']
    return pl.pallas_call(
        functools.partial(_edge_mlpN_body, col),
        grid=(CAP // EBLK,),
        out_shape=[jax.ShapeDtypeStruct((CAP, F), jnp.float32),
                   jax.ShapeDtypeStruct((CAP, F), jnp.bfloat16)],
        in_specs=[tile((EBLK, 256)), tile((EBLK, 128)), tile((EBLK, F)),
                  full((192, 128)), full((1, 128)), full((128, F)),
                  full((1, F)), full((1, F)), full((1, F))],
        out_specs=[tile((EBLK, F))] * 2,
    )(gsurf, ghl, he, e['w1'], e['b1'].reshape(1, -1), e['w2'],
      e['b2'].reshape(1, -1), e['ln_g'].reshape(1, -1),
      e['ln_b'].reshape(1, -1))


def _lig_node_body(hl_ref, agg_ref, w1, b1, w2, b2, g, b, out_ref):
    agg = (agg_ref[0, 0:LG, :].astype(jnp.float32) +
           agg_ref[1, 0:LG, :].astype(jnp.float32))
    hl = hl_ref[...]
    w = w1[...]
    x = (jnp.dot(hl, w[0:F], preferred_element_type=jnp.float32) +
         jnp.dot(agg, w[F:2 * F], preferred_element_type=jnp.float32) + b1[...])
    x = jnp.maximum(x, 0.0)
    x = jnp.dot(x, w2[...], preferred_element_type=jnp.float32) + b2[...]
    out_ref[...] = hl + _ln(x, g[...], b[...])


def _lig_node(hl, agg2, cv):
    full = lambda s: pl.BlockSpec(s, lambda: (0,) * len(s))
    n = cv['node']
    return pl.pallas_call(
        _lig_node_body,
        out_shape=jax.ShapeDtypeStruct((LG, F), jnp.float32),
        in_specs=[full((LG, F)), full((NC, LPAD, F)),
                  full((128, 128)), full((1, 128)), full((128, F)),
                  full((1, F)), full((1, F)), full((1, F))],
        out_specs=full((LG, F)),
    )(hl, agg2, n['w1'], n['b1'].reshape(1, -1), n['w2'],
      n['b2'].reshape(1, -1), n['ln_g'].reshape(1, -1),
      n['ln_b'].reshape(1, -1))


def _pos_out_body(hl_ref, lp_ref, w1, b1, w2, b2, out_ref):
    x = jnp.dot(hl_ref[...], w1[...], preferred_element_type=jnp.float32) + b1[...]
    x = jnp.maximum(x, 0.0)
    x = jnp.dot(x, w2[...], preferred_element_type=jnp.float32) + b2[...]
    out_ref[...] = x + lp_ref[...]


def _pos_out(hl, lp, p):
    full = lambda s: pl.BlockSpec(s, lambda: (0,) * len(s))
    m = p['pos_mlp']
    return pl.pallas_call(
        _pos_out_body,
        out_shape=jax.ShapeDtypeStruct((LG, 3), jnp.float32),
        in_specs=[full((LG, F)), full((LG, 3)), full((F, F)), full((1, F)),
                  full((F, 3)), full((1, 3))],
        out_specs=full((LG, 3)),
    )(hl, lp, m['w1'], m['b1'].reshape(1, -1), m['w2'], m['b2'].reshape(1, -1))


def kernel(surface_pos, init_ligand_pos, batch_surface, batch_ligand, time,
           params):
    p = params
    hl0 = _lig_prologue(init_ligand_pos, time, p)
    mask, hs0, hs1, hs2 = _surf_mask(surface_pos, batch_surface,
                                     init_ligand_pos, batch_ligand, p)

    flat = jnp.nonzero(mask.reshape(-1), size=CAP,
                       fill_value=S * LG)[0].astype(jnp.int32)
    valid = flat < S * LG
    src = jnp.where(valid, flat // LG, S).astype(jnp.int32)
    dst = jnp.where(valid, flat - (flat // LG) * LG, LG).astype(jnp.int32)
    src3 = src.reshape(NW, CPT // GCH, GCH)
    dst3 = dst.reshape(NW, CPT // GCH, GCH)

    bf = jnp.bfloat16
    pad_rows = lambda x, n: jnp.pad(x, ((0, n - x.shape[0]), (0, 128 - F)))

    def poslane(pos):
        hi = pos.astype(bf)
        lo = (pos - hi.astype(jnp.float32)).astype(bf)
        return jnp.pad(jnp.concatenate([hi, lo], axis=1), ((0, 0), (0, 10)))

    surf_tab = jax.lax.bitcast_convert_type(
        jnp.pad(
            jnp.concatenate([poslane(surface_pos),
                             hs0.astype(bf), hs1.astype(bf), hs2.astype(bf)],
                            axis=1),
            ((0, SPAD - S), (0, 48))).reshape(SPAD, 128, 2),
        jnp.int32)
    lig_tab = jnp.pad(
        jnp.concatenate([jnp.pad(init_ligand_pos, ((0, 0), (0, 13))), hl0],
                        axis=1),
        ((0, LPAD - LG), (0, 48)))

    gsurf = jax.lax.bitcast_convert_type(
        _sc_gather(surf_tab, src3), bf).reshape(CAP, 256)
    glig0 = _sc_gather(lig_tab, dst3)

    zeros = jnp.zeros((LPAD, F), jnp.bfloat16)
    he, he_bf = _edge_mlp0(gsurf, glig0, p['convs'][0], p)
    agg2 = _sc_scatter_add(he_bf, dst3, zeros)
    hl = _lig_node(hl0, agg2, p['convs'][0])
    for li in (1, 2):
        cv = p['convs'][li]
        ghl = _sc_gather(pad_rows(hl, LPAD), dst3)
        he, he_bf = _edge_mlpN(gsurf, ghl, he, cv, 16 + li * F)
        agg2 = _sc_scatter_add(he_bf, dst3, zeros)
        hl = _lig_node(hl, agg2, cv)

    return _pos_out(hl, init_ligand_pos, p)


# speedup vs baseline: 1.5295x; 1.5295x over previous
import functools

import jax
import jax.numpy as jnp
from jax import lax
from jax.experimental import pallas as pl
from jax.experimental.pallas import tpu as pltpu
from jax.experimental.pallas import tpu_sc as plsc

S = 8000
LG = 2000
F = 64
NC, NS = 2, 16
NW = NC * NS
CPT = 5120
CAP = NW * CPT
SPAD = 8192
LPAD = 2048
EBLK = 2048
STS = 400
LTS = 200
GCH = 128
WIN = 1024
WB = 256


def _mesh():
    return plsc.VectorSubcoreMesh(core_axis_name="c", subcore_axis_name="s")


def _sc_params():
    return pltpu.CompilerParams(use_tc_tiling_on_sc=False)


def _lig_prologue_body(lp_ref, t_ref, tw1, tb1, tw2, tb2, wl, bl, wg, bg, wb,
                       out_ref):
    t = t_ref[...]
    half = 32
    k = lax.broadcasted_iota(jnp.int32, (1, half), 1).astype(jnp.float32)
    freqs = jnp.exp(-jnp.log(10000.0) / (half - 1) * k)
    a = t * freqs
    ht = jnp.concatenate([jnp.sin(a), jnp.cos(a)], axis=1)
    x = jnp.dot(ht, tw1[...], preferred_element_type=jnp.float32) + tb1[...]
    c = 0.7978845608028654
    g = 0.5 * x * (1.0 + jnp.tanh(c * (x + 0.044715 * x * x * x)))
    ht = jnp.dot(g, tw2[...], preferred_element_type=jnp.float32) + tb2[...]
    lp = lp_ref[...]
    base = jnp.dot(lp, wl[...], preferred_element_type=jnp.float32) + bl[...]
    gate = jax.nn.sigmoid(
        jnp.dot(ht, wg[...], preferred_element_type=jnp.float32) + bg[...])
    out_ref[...] = base * gate + jnp.dot(ht, wb[...],
                                         preferred_element_type=jnp.float32)


def _lig_prologue(lp, t, p):
    tm, c = p['time_mlp'], p['csl']
    full = lambda s: pl.BlockSpec(s, lambda: (0,) * len(s))
    args = (lp, t,
            tm['w1'], tm['b1'].reshape(1, -1), tm['w2'], tm['b2'].reshape(1, -1),
            c['wl'], c['bl'].reshape(1, -1), c['wg'], c['bg'].reshape(1, -1),
            c['wb'])
    return pl.pallas_call(
        _lig_prologue_body,
        out_shape=jax.ShapeDtypeStruct((LG, F), jnp.float32),
        in_specs=[full(a.shape) for a in args],
        out_specs=full((LG, F)),
    )(*args)


def _ln(x, g, b):
    m = jnp.mean(x, axis=-1, keepdims=True)
    v = jnp.mean((x - m) ** 2, axis=-1, keepdims=True)
    return (x - m) * jax.lax.rsqrt(v + 1e-5) * g + b


def _surf_traj_body(sp_ref, sw, sb, nw1, nb1, nw2, nb2, ng, nbl,
                    hs0_ref, hs1_ref, hs2_ref):
    sp = sp_ref[...]
    hs = jnp.dot(sp, sw[...], preferred_element_type=jnp.float32) + sb[...]
    hs0_ref[...] = hs
    outs = (hs1_ref, hs2_ref)
    for li in range(2):
        up = jnp.maximum(
            jnp.dot(hs, nw1[li], preferred_element_type=jnp.float32) + nb1[li],
            0.0)
        up = jnp.dot(up, nw2[li], preferred_element_type=jnp.float32) + nb2[li]
        hs = hs + _ln(up, ng[li], nbl[li])
        outs[li][...] = hs


def _surf_traj(sp, p):
    nw1 = jnp.stack([cv['node']['w1'][:F] for cv in p['convs'][:2]])
    nb1 = jnp.stack([cv['node']['b1'].reshape(1, -1) for cv in p['convs'][:2]])
    nw2 = jnp.stack([cv['node']['w2'] for cv in p['convs'][:2]])
    nb2 = jnp.stack([cv['node']['b2'].reshape(1, -1) for cv in p['convs'][:2]])
    ng = jnp.stack([cv['node']['ln_g'].reshape(1, -1) for cv in p['convs'][:2]])
    nbl = jnp.stack([cv['node']['ln_b'].reshape(1, -1) for cv in p['convs'][:2]])
    tile = lambda s: pl.BlockSpec(s, lambda i: (i,) + (0,) * (len(s) - 1))
    full = lambda s: pl.BlockSpec(s, lambda i: (0,) * len(s))
    args = (sp, p['surf_enc']['w'], p['surf_enc']['b'].reshape(1, -1),
            nw1, nb1, nw2, nb2, ng, nbl)
    in_specs = [tile((STS, 3))] + [full(a.shape) for a in args[1:]]
    return pl.pallas_call(
        _surf_traj_body,
        grid=(S // STS,),
        out_shape=[jax.ShapeDtypeStruct((S, F), jnp.float32)] * 3,
        in_specs=in_specs,
        out_specs=[tile((STS, F))] * 3,
    )(*args)


def _maskT_body(lp_ref, bl_ref, spt_ref, bs_ref, mask_ref):
    lp = lp_ref[...]
    d2 = jnp.zeros((LTS, S), jnp.float32)
    for ci in range(3):
        diff = lp[:, ci:ci + 1] - spt_ref[ci:ci + 1, :]
        d2 = d2 + diff * diff
    same = bl_ref[...] == bs_ref[...]
    mask_ref[...] = jnp.where(same & (d2 < 9.0), jnp.int32(1), jnp.int32(0))


def _maskT(lp, bl, sp, bs):
    tile = lambda s: pl.BlockSpec(s, lambda i: (i,) + (0,) * (len(s) - 1))
    full = lambda s: pl.BlockSpec(s, lambda i: (0,) * len(s))
    return pl.pallas_call(
        _maskT_body,
        grid=(LG // LTS,),
        out_shape=jax.ShapeDtypeStruct((LG, S), jnp.int32),
        in_specs=[tile((LTS, 3)), tile((LTS, 1)), full((3, S)), full((1, S))],
        out_specs=tile((LTS, S)),
    )(lp, bl.reshape(LG, 1), sp.T, bs.reshape(1, S))


def _sc_gather(table, idx3):
    D = table.shape[1]
    dt = table.dtype
    iters = CPT // GCH
    NB = max(2, min(8, 400 * 1024 // (GCH * D * dt.itemsize)))

    @functools.partial(
        pl.kernel, mesh=_mesh(), compiler_params=_sc_params(),
        out_type=jax.ShapeDtypeStruct((CAP, D), dt),
        scratch_types=[pltpu.VMEM((CPT // GCH, GCH), jnp.int32),
                       pltpu.VMEM((NB, GCH, D), dt)] +
                      [pltpu.SemaphoreType.DMA] * (2 * NB))
    def k(tab_hbm, idx_hbm, out_hbm, idx_v, rows_v, *sems):
        gsem, wsem = sems[:NB], sems[NB:]
        wid = lax.axis_index("s") * NC + lax.axis_index("c")
        base = wid * CPT
        pltpu.sync_copy(idx_hbm.at[wid], idx_v)

        gh = [None] * NB
        wh = [None] * NB
        for i in range(iters + NB - 1):
            if i < iters:
                b = i % NB
                if wh[b] is not None:
                    wh[b].wait()
                gh[b] = pltpu.async_copy(
                    tab_hbm.at[idx_v.at[i]], rows_v.at[b], gsem[b])
            j = i - (NB - 1)
            if j >= 0:
                bj = j % NB
                gh[bj].wait()
                wh[bj] = pltpu.async_copy(
                    rows_v.at[bj], out_hbm.at[pl.ds(base + j * GCH, GCH)],
                    wsem[bj])
        for h in wh:
            if h is not None:
                h.wait()

    return k(table, idx3)


def _edge_mlp_common(gs, gl, he, w1, b1, w2, b2, g, b):
    bf = jnp.bfloat16
    w = w1[...]
    x = (jnp.dot(gs, w[0:F].astype(bf), preferred_element_type=jnp.float32) +
         jnp.dot(gl, w[F:2 * F].astype(bf),
                 preferred_element_type=jnp.float32) +
         jnp.dot(he.astype(bf), w[2 * F:3 * F].astype(bf),
                 preferred_element_type=jnp.float32)
         + b1[...])
    x = jnp.maximum(x, 0.0)
    x = jnp.dot(x.astype(bf), w2[...].astype(bf),
                preferred_element_type=jnp.float32) + b2[...]
    return he + _ln(x, g[...], b[...])


def _edge_layer_body(layer0, col, wl_ref, gsurf_ref, he_ref, dst_ref,
                     lw0, lw1, lw2, lw3, z0, z1, z2, z3,
                     w1, b1, w2, b2, g, b, emw, off,
                     he_out, a0, a1, a2, a3):
    f32 = jnp.float32
    bf = jnp.bfloat16
    i = pl.program_id(0)
    w0 = wl_ref[i] * WB
    dstv = dst_ref[...]
    iot = lax.broadcasted_iota(jnp.int32, (EBLK, WIN), 1) + w0
    ohb = (dstv == iot).astype(bf)
    ligwin = jnp.concatenate(
        [lw0[...], lw1[...], lw2[...], lw3[...]], axis=0)
    ex = jnp.dot(ohb, ligwin, preferred_element_type=f32)
    gl = ex[:, 16:16 + F].astype(bf)
    gs = gsurf_ref[:, col:col + F]

    if layer0:
        sp = gsurf_ref[:, 0:3].astype(f32) + gsurf_ref[:, 3:6].astype(f32)
        lp = ex[:, 0:3] + ex[:, 3:6]
        ev = sp - lp
        nrm = jnp.sqrt(jnp.sum(ev * ev, axis=1, keepdims=True))
        v = ev / (nrm + 1e-7)
        coeff = -0.5 / ((10.0 / 18.0) ** 2)
        sca = jnp.exp(coeff * (nrm - off[...]) ** 2)
        em = emw[...]
        he = jnp.concatenate(
            [sca] + [v[:, ci:ci + 1] * em for ci in range(3)], axis=1)
    else:
        he = he_ref[...]

    he_new = _edge_mlp_common(gs, gl, he, w1, b1, w2, b2, g, b)
    he_out[...] = he_new
    hb = he_new.astype(bf)

    prev = wl_ref[jnp.maximum(i - 1, 0)]
    first = (i == 0) | (wl_ref[i] != prev)
    for k, ak in enumerate((a0, a1, a2, a3)):
        pk = jax.lax.dot_general(ohb[:, k * WB:(k + 1) * WB], hb,
                                 (((0,), (0,)), ((), ())),
                                 preferred_element_type=f32)

        @pl.when(first)
        def _():
            ak[...] = pk

        @pl.when(jnp.logical_not(first))
        def _():
            ak[...] += pk


def _edge_layer(layer0, col, gsurf, he, dst2d, ligw, wl, cv, p):
    e = cv['edge']
    tile = lambda s: pl.BlockSpec(s, lambda i, wl: (i,) + (0,) * (len(s) - 1))
    full = lambda s: pl.BlockSpec(s, lambda i, wl: (0,) * len(s))

    def wspec(k):
        return pl.BlockSpec((WB, 80), lambda i, wl, k=k: (wl[i] + k, 0))

    def aspec(k):
        return pl.BlockSpec((WB, F), lambda i, wl, k=k: (wl[i] + k, 0))

    zeros = [jnp.zeros((LPAD, F), jnp.float32) for _ in range(4)]
    in_specs = [tile((EBLK, 208)), tile((EBLK, F)), tile((EBLK, 1)),
                wspec(0), wspec(1), wspec(2), wspec(3),
                aspec(0), aspec(1), aspec(2), aspec(3),
                full((192, 128)), full((1, 128)), full((128, F)),
                full((1, F)), full((1, F)), full((1, F)),
                full((1, 15)), full((1, 19))]
    out_specs = [tile((EBLK, F)), aspec(0), aspec(1), aspec(2), aspec(3)]
    grid_spec = pltpu.PrefetchScalarGridSpec(
        num_scalar_prefetch=1, grid=(CAP // EBLK,),
        in_specs=in_specs, out_specs=out_specs)
    return pl.pallas_call(
        functools.partial(_edge_layer_body, layer0, col),
        grid_spec=grid_spec,
        out_shape=[jax.ShapeDtypeStruct((CAP, F), jnp.float32)] +
                  [jax.ShapeDtypeStruct((LPAD, F), jnp.float32)] * 4,
        input_output_aliases={8: 1, 9: 2, 10: 3, 11: 4},
    )(wl, gsurf, he, dst2d, ligw[0], ligw[1], ligw[2], ligw[3],
      zeros[0], zeros[1], zeros[2], zeros[3],
      e['w1'], e['b1'].reshape(1, -1), e['w2'], e['b2'].reshape(1, -1),
      e['ln_g'].reshape(1, -1), e['ln_b'].reshape(1, -1),
      p['edge_map_w'], p['gs_offset'].reshape(1, 19))


def _lig_node_body(hl_ref, a0, a1, a2, a3, w1, b1, w2, b2, g, b, out_ref):
    agg = (a0[0:LG, :] + a1[0:LG, :]) + (a2[0:LG, :] + a3[0:LG, :])
    hl = hl_ref[...]
    w = w1[...]
    x = (jnp.dot(hl, w[0:F], preferred_element_type=jnp.float32) +
         jnp.dot(agg, w[F:2 * F], preferred_element_type=jnp.float32) + b1[...])
    x = jnp.maximum(x, 0.0)
    x = jnp.dot(x, w2[...], preferred_element_type=jnp.float32) + b2[...]
    out_ref[...] = hl + _ln(x, g[...], b[...])


def _lig_node(hl, aggs, cv):
    full = lambda s: pl.BlockSpec(s, lambda: (0,) * len(s))
    n = cv['node']
    return pl.pallas_call(
        _lig_node_body,
        out_shape=jax.ShapeDtypeStruct((LG, F), jnp.float32),
        in_specs=[full((LG, F))] + [full((LPAD, F))] * 4 + [
            full((128, 128)), full((1, 128)), full((128, F)),
            full((1, F)), full((1, F)), full((1, F))],
        out_specs=full((LG, F)),
    )(hl, *aggs, n['w1'], n['b1'].reshape(1, -1), n['w2'],
      n['b2'].reshape(1, -1), n['ln_g'].reshape(1, -1),
      n['ln_b'].reshape(1, -1))


def _pos_out_body(hl_ref, lp_ref, w1, b1, w2, b2, out_ref):
    x = jnp.dot(hl_ref[...], w1[...], preferred_element_type=jnp.float32) + b1[...]
    x = jnp.maximum(x, 0.0)
    x = jnp.dot(x, w2[...], preferred_element_type=jnp.float32) + b2[...]
    out_ref[...] = x + lp_ref[...]


def _pos_out(hl, lp, p):
    full = lambda s: pl.BlockSpec(s, lambda: (0,) * len(s))
    m = p['pos_mlp']
    return pl.pallas_call(
        _pos_out_body,
        out_shape=jax.ShapeDtypeStruct((LG, 3), jnp.float32),
        in_specs=[full((LG, F)), full((LG, 3)), full((F, F)), full((1, F)),
                  full((F, 3)), full((1, 3))],
        out_specs=full((LG, 3)),
    )(hl, lp, m['w1'], m['b1'].reshape(1, -1), m['w2'], m['b2'].reshape(1, -1))


def kernel(surface_pos, init_ligand_pos, batch_surface, batch_ligand, time,
           params):
    p = params
    bf = jnp.bfloat16
    hl0 = _lig_prologue(init_ligand_pos, time, p)
    hs0, hs1, hs2 = _surf_traj(surface_pos, p)
    maskT = _maskT(init_ligand_pos, batch_ligand, surface_pos, batch_surface)

    flat = jnp.nonzero(maskT.reshape(-1), size=CAP,
                       fill_value=LG * S)[0].astype(jnp.int32)
    valid = flat < LG * S
    dst = (flat // S).astype(jnp.int32)
    src = jnp.where(valid, flat - dst * S, S).astype(jnp.int32)
    src3 = src.reshape(NW, CPT // GCH, GCH)
    dst2d = dst.reshape(CAP, 1)
    wl = jnp.clip(dst.reshape(CAP // EBLK, EBLK)[:, 0] // WB,
                  0, (LPAD - WIN) // WB).astype(jnp.int32)

    def poslane(pos):
        hi = pos.astype(bf)
        lo = (pos - hi.astype(jnp.float32)).astype(bf)
        return jnp.pad(jnp.concatenate([hi, lo], axis=1), ((0, 0), (0, 10)))

    surf_tab = jnp.pad(
        jnp.concatenate([poslane(surface_pos),
                         hs0.astype(bf), hs1.astype(bf), hs2.astype(bf)],
                        axis=1),
        ((0, SPAD - S), (0, 0)))
    gsurf = _sc_gather(surf_tab, src3)

    lpl = poslane(init_ligand_pos)
    he = jnp.zeros((CAP, F), jnp.float32)
    hl = hl0
    for li in range(3):
        cv = p['convs'][li]
        ligw = jnp.pad(jnp.concatenate([lpl, hl.astype(bf)], axis=1),
                       ((0, LPAD - LG), (0, 0)))
        lw = [ligw] * 4
        outs = _edge_layer(li == 0, 16 + li * F, gsurf, he, dst2d, lw,
                           wl, cv, p)
        he, aggs = outs[0], outs[1:]
        hl = _lig_node(hl, aggs, cv)

    return _pos_out(hl, init_ligand_pos, p)


# speedup vs baseline: 1.5384x; 1.0059x over previous
import functools

import jax
import jax.numpy as jnp
from jax import lax
from jax.experimental import pallas as pl
from jax.experimental.pallas import tpu as pltpu
from jax.experimental.pallas import tpu_sc as plsc

S = 8000
LG = 2000
F = 64
NC, NS = 2, 16
NW = NC * NS
CPT = 5120
CAP = NW * CPT
SPAD = 8192
LPAD = 2048
EBLK = 2048
STS = 400
LTS = 200
GCH = 128
WIN = 1024
WB = 256


def _mesh():
    return plsc.VectorSubcoreMesh(core_axis_name="c", subcore_axis_name="s")


def _sc_params():
    return pltpu.CompilerParams(use_tc_tiling_on_sc=False)


def _lig_prologue_body(lp_ref, t_ref, tw1, tb1, tw2, tb2, wl, bl, wg, bg, wb,
                       out_ref):
    t = t_ref[...]
    half = 32
    k = lax.broadcasted_iota(jnp.int32, (1, half), 1).astype(jnp.float32)
    freqs = jnp.exp(-jnp.log(10000.0) / (half - 1) * k)
    a = t * freqs
    ht = jnp.concatenate([jnp.sin(a), jnp.cos(a)], axis=1)
    x = jnp.dot(ht, tw1[...], preferred_element_type=jnp.float32) + tb1[...]
    c = 0.7978845608028654
    g = 0.5 * x * (1.0 + jnp.tanh(c * (x + 0.044715 * x * x * x)))
    ht = jnp.dot(g, tw2[...], preferred_element_type=jnp.float32) + tb2[...]
    lp = lp_ref[...]
    base = jnp.dot(lp, wl[...], preferred_element_type=jnp.float32) + bl[...]
    gate = jax.nn.sigmoid(
        jnp.dot(ht, wg[...], preferred_element_type=jnp.float32) + bg[...])
    out_ref[...] = base * gate + jnp.dot(ht, wb[...],
                                         preferred_element_type=jnp.float32)


def _lig_prologue(lp, t, p):
    tm, c = p['time_mlp'], p['csl']
    full = lambda s: pl.BlockSpec(s, lambda: (0,) * len(s))
    args = (lp, t,
            tm['w1'], tm['b1'].reshape(1, -1), tm['w2'], tm['b2'].reshape(1, -1),
            c['wl'], c['bl'].reshape(1, -1), c['wg'], c['bg'].reshape(1, -1),
            c['wb'])
    return pl.pallas_call(
        _lig_prologue_body,
        out_shape=jax.ShapeDtypeStruct((LG, F), jnp.float32),
        in_specs=[full(a.shape) for a in args],
        out_specs=full((LG, F)),
    )(*args)


def _ln(x, g, b):
    m = jnp.mean(x, axis=-1, keepdims=True)
    v = jnp.mean((x - m) ** 2, axis=-1, keepdims=True)
    return (x - m) * jax.lax.rsqrt(v + 1e-5) * g + b


def _surf_traj_body(sp_ref, sw, sb, nw1, nb1, nw2, nb2, ng, nbl,
                    hs0_ref, hs1_ref, hs2_ref):
    sp = sp_ref[...]
    hs = jnp.dot(sp, sw[...], preferred_element_type=jnp.float32) + sb[...]
    hs0_ref[...] = hs
    outs = (hs1_ref, hs2_ref)
    for li in range(2):
        up = jnp.maximum(
            jnp.dot(hs, nw1[li], preferred_element_type=jnp.float32) + nb1[li],
            0.0)
        up = jnp.dot(up, nw2[li], preferred_element_type=jnp.float32) + nb2[li]
        hs = hs + _ln(up, ng[li], nbl[li])
        outs[li][...] = hs


def _surf_traj(sp, p):
    nw1 = jnp.stack([cv['node']['w1'][:F] for cv in p['convs'][:2]])
    nb1 = jnp.stack([cv['node']['b1'].reshape(1, -1) for cv in p['convs'][:2]])
    nw2 = jnp.stack([cv['node']['w2'] for cv in p['convs'][:2]])
    nb2 = jnp.stack([cv['node']['b2'].reshape(1, -1) for cv in p['convs'][:2]])
    ng = jnp.stack([cv['node']['ln_g'].reshape(1, -1) for cv in p['convs'][:2]])
    nbl = jnp.stack([cv['node']['ln_b'].reshape(1, -1) for cv in p['convs'][:2]])
    tile = lambda s: pl.BlockSpec(s, lambda i: (i,) + (0,) * (len(s) - 1))
    full = lambda s: pl.BlockSpec(s, lambda i: (0,) * len(s))
    args = (sp, p['surf_enc']['w'], p['surf_enc']['b'].reshape(1, -1),
            nw1, nb1, nw2, nb2, ng, nbl)
    in_specs = [tile((STS, 3))] + [full(a.shape) for a in args[1:]]
    return pl.pallas_call(
        _surf_traj_body,
        grid=(S // STS,),
        out_shape=[jax.ShapeDtypeStruct((S, F), jnp.float32)] * 3,
        in_specs=in_specs,
        out_specs=[tile((STS, F))] * 3,
    )(*args)


def _maskT_body(lp_ref, bl_ref, spt_ref, bs_ref, mask_ref):
    lp = lp_ref[...]
    d2 = jnp.zeros((LTS, S), jnp.float32)
    for ci in range(3):
        diff = lp[:, ci:ci + 1] - spt_ref[ci:ci + 1, :]
        d2 = d2 + diff * diff
    same = bl_ref[...] == bs_ref[...]
    mask_ref[...] = jnp.where(same & (d2 < 9.0), jnp.int32(1), jnp.int32(0))


def _maskT(lp, bl, sp, bs):
    tile = lambda s: pl.BlockSpec(s, lambda i: (i,) + (0,) * (len(s) - 1))
    full = lambda s: pl.BlockSpec(s, lambda i: (0,) * len(s))
    return pl.pallas_call(
        _maskT_body,
        grid=(LG // LTS,),
        out_shape=jax.ShapeDtypeStruct((LG, S), jnp.int32),
        in_specs=[tile((LTS, 3)), tile((LTS, 1)), full((3, S)), full((1, S))],
        out_specs=tile((LTS, S)),
    )(lp, bl.reshape(LG, 1), sp.T, bs.reshape(1, S))


def _sc_gather(table, idx3):
    D = table.shape[1]
    dt = table.dtype
    iters = CPT // GCH
    NB = max(2, min(8, 400 * 1024 // (GCH * D * dt.itemsize)))

    @functools.partial(
        pl.kernel, mesh=_mesh(), compiler_params=_sc_params(),
        out_type=jax.ShapeDtypeStruct((CAP, D), dt),
        scratch_types=[pltpu.VMEM((CPT // GCH, GCH), jnp.int32),
                       pltpu.VMEM((NB, GCH, D), dt)] +
                      [pltpu.SemaphoreType.DMA] * (2 * NB))
    def k(tab_hbm, idx_hbm, out_hbm, idx_v, rows_v, *sems):
        gsem, wsem = sems[:NB], sems[NB:]
        wid = lax.axis_index("s") * NC + lax.axis_index("c")
        base = wid * CPT
        pltpu.sync_copy(idx_hbm.at[wid], idx_v)

        gh = [None] * NB
        wh = [None] * NB
        for i in range(iters + NB - 1):
            if i < iters:
                b = i % NB
                if wh[b] is not None:
                    wh[b].wait()
                gh[b] = pltpu.async_copy(
                    tab_hbm.at[idx_v.at[i]], rows_v.at[b], gsem[b])
            j = i - (NB - 1)
            if j >= 0:
                bj = j % NB
                gh[bj].wait()
                wh[bj] = pltpu.async_copy(
                    rows_v.at[bj], out_hbm.at[pl.ds(base + j * GCH, GCH)],
                    wsem[bj])
        for h in wh:
            if h is not None:
                h.wait()

    return k(table, idx3)


def _edge_mlp_common(gs, gl, he, w1, b1, w2, b2, g, b):
    bf = jnp.bfloat16
    w = w1[...]
    x = (jnp.dot(gs, w[0:F].astype(bf), preferred_element_type=jnp.float32) +
         jnp.dot(gl, w[F:2 * F].astype(bf),
                 preferred_element_type=jnp.float32) +
         jnp.dot(he.astype(bf), w[2 * F:3 * F].astype(bf),
                 preferred_element_type=jnp.float32)
         + b1[...])
    x = jnp.maximum(x, 0.0)
    x = jnp.dot(x.astype(bf), w2[...].astype(bf),
                preferred_element_type=jnp.float32) + b2[...]
    return he + _ln(x, g[...], b[...])


def _edge_layer_body(layer0, col, wl_ref, gsurf_ref, he_ref, dst_ref,
                     dstT_ref, lw0, lw1, lw2, lw3, z0, z1, z2, z3,
                     w1, b1, w2, b2, g, b, emw, off,
                     he_out, a0, a1, a2, a3):
    f32 = jnp.float32
    bf = jnp.bfloat16
    i = pl.program_id(0)
    w0 = wl_ref[i] * WB
    dstv = dst_ref[...]
    iot = lax.broadcasted_iota(jnp.int32, (EBLK, WIN), 1) + w0
    ohb = (dstv == iot).astype(bf)
    ligwin = jnp.concatenate(
        [lw0[...], lw1[...], lw2[...], lw3[...]], axis=0)
    ex = jnp.dot(ohb, ligwin, preferred_element_type=f32)
    gl = ex[:, 16:16 + F].astype(bf)
    gs = gsurf_ref[:, col:col + F]

    if layer0:
        sp = gsurf_ref[:, 0:3].astype(f32) + gsurf_ref[:, 3:6].astype(f32)
        lp = ex[:, 0:3] + ex[:, 3:6]
        ev = sp - lp
        nrm = jnp.sqrt(jnp.sum(ev * ev, axis=1, keepdims=True))
        v = ev / (nrm + 1e-7)
        coeff = -0.5 / ((10.0 / 18.0) ** 2)
        sca = jnp.exp(coeff * (nrm - off[...]) ** 2)
        em = emw[...]
        he = jnp.concatenate(
            [sca] + [v[:, ci:ci + 1] * em for ci in range(3)], axis=1)
    else:
        he = he_ref[...]

    he_new = _edge_mlp_common(gs, gl, he, w1, b1, w2, b2, g, b)
    he_out[...] = he_new
    hb = he_new.astype(bf)

    prev = wl_ref[jnp.maximum(i - 1, 0)]
    first = (i == 0) | (wl_ref[i] != prev)
    iotT = lax.broadcasted_iota(jnp.int32, (WIN, EBLK), 0) + w0
    ohbT = (iotT == dstT_ref[...]).astype(bf)
    for k, ak in enumerate((a0, a1, a2, a3)):
        pk = jnp.dot(ohbT[k * WB:(k + 1) * WB, :], hb,
                     preferred_element_type=f32)

        @pl.when(first)
        def _():
            ak[...] = pk

        @pl.when(jnp.logical_not(first))
        def _():
            ak[...] += pk


def _edge_layer(layer0, col, gsurf, he, dst2d, dstT, ligw, wl, cv, p):
    e = cv['edge']
    tile = lambda s: pl.BlockSpec(s, lambda i, wl: (i,) + (0,) * (len(s) - 1))
    full = lambda s: pl.BlockSpec(s, lambda i, wl: (0,) * len(s))

    def wspec(k):
        return pl.BlockSpec((WB, 80), lambda i, wl, k=k: (wl[i] + k, 0))

    def aspec(k):
        return pl.BlockSpec((WB, F), lambda i, wl, k=k: (wl[i] + k, 0))

    zeros = [jnp.zeros((LPAD, F), jnp.float32) for _ in range(4)]
    dtt = lambda: pl.BlockSpec((1, EBLK), lambda i, wl: (0, i))
    in_specs = [tile((EBLK, 208)), tile((EBLK, F)), tile((EBLK, 1)), dtt(),
                wspec(0), wspec(1), wspec(2), wspec(3),
                aspec(0), aspec(1), aspec(2), aspec(3),
                full((192, 128)), full((1, 128)), full((128, F)),
                full((1, F)), full((1, F)), full((1, F)),
                full((1, 15)), full((1, 19))]
    out_specs = [tile((EBLK, F)), aspec(0), aspec(1), aspec(2), aspec(3)]
    grid_spec = pltpu.PrefetchScalarGridSpec(
        num_scalar_prefetch=1, grid=(CAP // EBLK,),
        in_specs=in_specs, out_specs=out_specs)
    return pl.pallas_call(
        functools.partial(_edge_layer_body, layer0, col),
        grid_spec=grid_spec,
        out_shape=[jax.ShapeDtypeStruct((CAP, F), jnp.float32)] +
                  [jax.ShapeDtypeStruct((LPAD, F), jnp.float32)] * 4,
        input_output_aliases={9: 1, 10: 2, 11: 3, 12: 4},
    )(wl, gsurf, he, dst2d, dstT, ligw[0], ligw[1], ligw[2], ligw[3],
      zeros[0], zeros[1], zeros[2], zeros[3],
      e['w1'], e['b1'].reshape(1, -1), e['w2'], e['b2'].reshape(1, -1),
      e['ln_g'].reshape(1, -1), e['ln_b'].reshape(1, -1),
      p['edge_map_w'], p['gs_offset'].reshape(1, 19))


def _lig_node_body(hl_ref, a0, a1, a2, a3, w1, b1, w2, b2, g, b, out_ref):
    agg = (a0[0:LG, :] + a1[0:LG, :]) + (a2[0:LG, :] + a3[0:LG, :])
    hl = hl_ref[...]
    w = w1[...]
    x = (jnp.dot(hl, w[0:F], preferred_element_type=jnp.float32) +
         jnp.dot(agg, w[F:2 * F], preferred_element_type=jnp.float32) + b1[...])
    x = jnp.maximum(x, 0.0)
    x = jnp.dot(x, w2[...], preferred_element_type=jnp.float32) + b2[...]
    out_ref[...] = hl + _ln(x, g[...], b[...])


def _lig_node(hl, aggs, cv):
    full = lambda s: pl.BlockSpec(s, lambda: (0,) * len(s))
    n = cv['node']
    return pl.pallas_call(
        _lig_node_body,
        out_shape=jax.ShapeDtypeStruct((LG, F), jnp.float32),
        in_specs=[full((LG, F))] + [full((LPAD, F))] * 4 + [
            full((128, 128)), full((1, 128)), full((128, F)),
            full((1, F)), full((1, F)), full((1, F))],
        out_specs=full((LG, F)),
    )(hl, *aggs, n['w1'], n['b1'].reshape(1, -1), n['w2'],
      n['b2'].reshape(1, -1), n['ln_g'].reshape(1, -1),
      n['ln_b'].reshape(1, -1))


def _pos_out_body(hl_ref, lp_ref, w1, b1, w2, b2, out_ref):
    x = jnp.dot(hl_ref[...], w1[...], preferred_element_type=jnp.float32) + b1[...]
    x = jnp.maximum(x, 0.0)
    x = jnp.dot(x, w2[...], preferred_element_type=jnp.float32) + b2[...]
    out_ref[...] = x + lp_ref[...]


def _pos_out(hl, lp, p):
    full = lambda s: pl.BlockSpec(s, lambda: (0,) * len(s))
    m = p['pos_mlp']
    return pl.pallas_call(
        _pos_out_body,
        out_shape=jax.ShapeDtypeStruct((LG, 3), jnp.float32),
        in_specs=[full((LG, F)), full((LG, 3)), full((F, F)), full((1, F)),
                  full((F, 3)), full((1, 3))],
        out_specs=full((LG, 3)),
    )(hl, lp, m['w1'], m['b1'].reshape(1, -1), m['w2'], m['b2'].reshape(1, -1))


def kernel(surface_pos, init_ligand_pos, batch_surface, batch_ligand, time,
           params):
    p = params
    bf = jnp.bfloat16
    hl0 = _lig_prologue(init_ligand_pos, time, p)
    hs0, hs1, hs2 = _surf_traj(surface_pos, p)
    maskT = _maskT(init_ligand_pos, batch_ligand, surface_pos, batch_surface)

    flat = jnp.nonzero(maskT.reshape(-1), size=CAP,
                       fill_value=LG * S)[0].astype(jnp.int32)
    valid = flat < LG * S
    dst = (flat // S).astype(jnp.int32)
    src = jnp.where(valid, flat - dst * S, S).astype(jnp.int32)
    src3 = src.reshape(NW, CPT // GCH, GCH)
    dst2d = dst.reshape(CAP, 1)
    dstT = dst.reshape(1, CAP)
    wl = jnp.clip(dst.reshape(CAP // EBLK, EBLK)[:, 0] // WB,
                  0, (LPAD - WIN) // WB).astype(jnp.int32)

    def poslane(pos):
        hi = pos.astype(bf)
        lo = (pos - hi.astype(jnp.float32)).astype(bf)
        return jnp.pad(jnp.concatenate([hi, lo], axis=1), ((0, 0), (0, 10)))

    surf_tab = jnp.pad(
        jnp.concatenate([poslane(surface_pos),
                         hs0.astype(bf), hs1.astype(bf), hs2.astype(bf)],
                        axis=1),
        ((0, SPAD - S), (0, 0)))
    gsurf = _sc_gather(surf_tab, src3)

    lpl = poslane(init_ligand_pos)
    he = jnp.zeros((CAP, F), jnp.float32)
    hl = hl0
    for li in range(3):
        cv = p['convs'][li]
        ligw = jnp.pad(jnp.concatenate([lpl, hl.astype(bf)], axis=1),
                       ((0, LPAD - LG), (0, 0)))
        lw = [ligw] * 4
        outs = _edge_layer(li == 0, 16 + li * F, gsurf, he, dst2d, dstT,
                           lw, wl, cv, p)
        he, aggs = outs[0], outs[1:]
        hl = _lig_node(hl, aggs, cv)

    return _pos_out(hl, init_ligand_pos, p)
